# Initial kernel scaffold; baseline (speedup 1.0000x reference)
import jax
import jax.numpy as jnp
from jax.experimental import pallas as pl


def kernel(x, table):
    raise NotImplementedError("write your pallas kernel here")


# speedup vs baseline: 1.4952x; 1.4952x over previous
import functools

import jax
import jax.numpy as jnp
from jax import lax
from jax.experimental import pallas as pl
from jax.experimental.pallas import tpu as pltpu
from jax.experimental.pallas import tpu_sc as plsc

_B, _L = 4096, 200
_D = 32
_N = _B * _L


def _make_gather(n_total: int, d: int, chunk: int):
    info = plsc.get_sparse_core_info()
    nc, ns = info.num_cores, info.num_subcores
    nw = nc * ns
    per_w = n_total // nw
    n_chunks = per_w // chunk
    assert per_w % chunk == 0 and n_total % nw == 0

    mesh = plsc.VectorSubcoreMesh(core_axis_name="c", subcore_axis_name="s")

    @functools.partial(
        pl.kernel,
        mesh=mesh,
        out_type=jax.ShapeDtypeStruct((n_total, d), jnp.float32),
        scratch_types=[
            pltpu.VMEM((chunk,), jnp.int32),
            pltpu.VMEM((chunk, d), jnp.float32),
            pltpu.SemaphoreType.DMA,
        ],
        compiler_params=pltpu.CompilerParams(use_tc_tiling_on_sc=False),
    )
    def k(idx_hbm, table_hbm, out_hbm, idx_v, rows_v, sem):
        wid = lax.axis_index("s") * nc + lax.axis_index("c")
        base = wid * per_w

        def body(i, carry):
            off = base + i * chunk
            pltpu.sync_copy(idx_hbm.at[pl.ds(off, chunk)], idx_v)
            pltpu.async_copy(table_hbm.at[idx_v], rows_v, sem).wait()
            pltpu.sync_copy(rows_v, out_hbm.at[pl.ds(off, chunk)])
            return carry

        lax.fori_loop(0, n_chunks, body, 0)

    return k


_gather = _make_gather(_N, _D, chunk=3200)


@jax.jit
def kernel(x, table):
    idx = x.reshape(-1).astype(jnp.int32)
    out = _gather(idx, table)
    return out.reshape(_B, _L, _D)


# speedup vs baseline: 1.5009x; 1.0038x over previous
import functools

import jax
import jax.numpy as jnp
from jax import lax
from jax.experimental import pallas as pl
from jax.experimental.pallas import tpu as pltpu
from jax.experimental.pallas import tpu_sc as plsc

_B, _L = 4096, 200
_D = 32
_N = _B * _L


def _make_gather(n_total: int, d: int, chunk: int):
    info = plsc.get_sparse_core_info()
    nc, ns = info.num_cores, info.num_subcores
    nw = nc * ns
    per_w = n_total // nw
    n_chunks = per_w // chunk
    assert per_w % chunk == 0 and n_total % nw == 0 and n_chunks >= 2

    mesh = plsc.VectorSubcoreMesh(core_axis_name="c", subcore_axis_name="s")

    @functools.partial(
        pl.kernel,
        mesh=mesh,
        out_type=jax.ShapeDtypeStruct((n_total, d), jnp.float32),
        scratch_types=[
            pltpu.VMEM((4, chunk), jnp.int32),
            pltpu.VMEM((2, chunk, d), jnp.float32),
            pltpu.SemaphoreType.DMA((4,)),
            pltpu.SemaphoreType.DMA((2,)),
            pltpu.SemaphoreType.DMA((2,)),
        ],
        compiler_params=pltpu.CompilerParams(use_tc_tiling_on_sc=False),
    )
    def k(idx_hbm, table_hbm, out_hbm, idx_v, rows_v, isem, gsem, osem):
        wid = lax.axis_index("s") * nc + lax.axis_index("c")
        base = wid * per_w

        def start_idx(i):
            return pltpu.async_copy(
                idx_hbm.at[pl.ds(base + i * chunk, chunk)],
                idx_v.at[i % 4],
                isem.at[i % 4],
            )

        def start_gather(i):
            return pltpu.async_copy(
                table_hbm.at[idx_v.at[i % 4]],
                rows_v.at[i % 2],
                gsem.at[i % 2],
            )

        def start_out(i):
            return pltpu.async_copy(
                rows_v.at[i % 2],
                out_hbm.at[pl.ds(base + i * chunk, chunk)],
                osem.at[i % 2],
            )

        idx_cp = {}
        g_cp = {}
        o_cp = {}
        idx_cp[0] = start_idx(0)
        idx_cp[1] = start_idx(1)
        for i in range(n_chunks):
            idx_cp.pop(i).wait()
            if i >= 2:
                o_cp.pop(i - 2).wait()
            g_cp[i] = start_gather(i)
            if i + 2 < n_chunks:
                idx_cp[i + 2] = start_idx(i + 2)
            if i >= 1:
                g_cp.pop(i - 1).wait()
                o_cp[i - 1] = start_out(i - 1)
        g_cp.pop(n_chunks - 1).wait()
        o_cp[n_chunks - 1] = start_out(n_chunks - 1)
        o_cp.pop(n_chunks - 2).wait()
        o_cp.pop(n_chunks - 1).wait()

    return k


_gather = _make_gather(_N, _D, chunk=1600)


@jax.jit
def kernel(x, table):
    idx = x.reshape(-1).astype(jnp.int32)
    out = _gather(idx, table)
    return out.reshape(_B, _L, _D)
